# baseline (speedup 1.0000x reference)
import functools

import jax
import jax.numpy as jnp
from jax import lax
from jax.experimental import pallas as pl
from jax.experimental.pallas import tpu as pltpu
from jax.experimental.pallas import tpu_sc as plsc

VOCAB = 1000000
D = 64
B = 16384
NEG = 20
PAIR = 128

NC = 2
NS = 16
NW = NC * NS
BPW = B // NW
CHUNK = 32
NCHUNK = BPW // CHUNK
JH = NEG // 2


def _sc_scores(cen_idx, ctx_idx, neg_idx, cen_pairs, ctx_pairs):
    mesh = plsc.VectorSubcoreMesh(
        core_axis_name="c", subcore_axis_name="s",
        num_cores=NC, num_subcores=NS,
    )

    @functools.partial(
        pl.kernel,
        out_type=jax.ShapeDtypeStruct((B, 128), jnp.float32),
        mesh=mesh,
        scratch_types=[
            pltpu.VMEM((BPW,), jnp.int32),
            pltpu.VMEM((BPW,), jnp.int32),
            pltpu.VMEM((BPW,), jnp.int32),
            pltpu.VMEM((BPW,), jnp.int32),
            pltpu.VMEM((BPW * NEG,), jnp.int32),
            pltpu.VMEM((BPW * NEG,), jnp.int32),
            pltpu.VMEM((CHUNK, PAIR), jnp.float32),
            pltpu.VMEM((CHUNK, PAIR), jnp.float32),
            pltpu.VMEM((CHUNK * NEG, PAIR), jnp.float32),
            pltpu.VMEM((CHUNK, 128), jnp.float32),
            pltpu.SemaphoreType.DMA,
        ],
        compiler_params=pltpu.CompilerParams(
            use_tc_tiling_on_sc=True, needs_layout_passes=False),
    )
    def body(cen_idx_h, ctx_idx_h, neg_idx_h, cen_emb_h, ctx_emb_h, out_h,
             cq_v, cpo_v, xq_v, xpo_v, nq_v, npo_v,
             cen_v, ctx_v, neg_v, sc_v, sem):
        wid = lax.axis_index("s") * NC + lax.axis_index("c")
        base = wid * BPW
        pltpu.sync_copy(cen_idx_h.at[pl.ds(base, BPW)], cq_v)
        pltpu.sync_copy(ctx_idx_h.at[pl.ds(base, BPW)], xq_v)
        pltpu.sync_copy(neg_idx_h.at[pl.ds(base * NEG, BPW * NEG)], nq_v)

        def split_cx(i, carry):
            c = cq_v[pl.ds(i * 16, 16)]
            x = xq_v[pl.ds(i * 16, 16)]
            cpo_v[pl.ds(i * 16, 16)] = (c & 1) * D
            xpo_v[pl.ds(i * 16, 16)] = (x & 1) * D
            cq_v[pl.ds(i * 16, 16)] = c >> 1
            xq_v[pl.ds(i * 16, 16)] = x >> 1
            return carry

        lax.fori_loop(0, BPW // 16, split_cx, 0)

        def split_n(i, carry):
            v = nq_v[pl.ds(i * 16, 16)]
            npo_v[pl.ds(i * 16, 16)] = (v & 1) * D
            nq_v[pl.ds(i * 16, 16)] = v >> 1
            return carry

        lax.fori_loop(0, BPW * NEG // 16, split_n, 0)

        lanes = lax.iota(jnp.int32, 16)

        def chunk(g, carry):
            c1 = pltpu.async_copy(
                cen_emb_h.at[cq_v.at[pl.ds(g * CHUNK, CHUNK)]], cen_v, sem)
            c2 = pltpu.async_copy(
                ctx_emb_h.at[xq_v.at[pl.ds(g * CHUNK, CHUNK)]], ctx_v, sem)
            c3 = pltpu.async_copy(
                ctx_emb_h.at[nq_v.at[pl.ds(g * CHUNK * NEG, CHUNK * NEG)]],
                neg_v, sem)
            c1.wait()
            c2.wait()
            c3.wait()

            def group(t, gcarry):
                rvec = t * 16 + lanes
                cpar = cpo_v[pl.ds(g * CHUNK + t * 16, 16)]
                xpar = xpo_v[pl.ds(g * CHUNK + t * 16, 16)]
                nrow = rvec * NEG
                mbase = (g * CHUNK + t * 16) * NEG + lanes * NEG

                accs = [jnp.zeros((16,), jnp.float32) for _ in range(NEG + 1)]
                for half in range(2):
                    js = range(half * JH, (half + 1) * JH)
                    npars = [plsc.load_gather(npo_v, [mbase + j]) for j in js]

                    def dstep(d, dc):
                        a = list(dc)
                        c = plsc.load_gather(cen_v, [rvec, cpar + d])
                        if half == 0:
                            x = plsc.load_gather(ctx_v, [rvec, xpar + d])
                            a[0] = a[0] + c * x
                        for k, j in enumerate(js):
                            n = plsc.load_gather(
                                neg_v, [nrow + j, npars[k] + d])
                            a[k + 1] = a[k + 1] + c * n
                        return tuple(a)

                    sel = [accs[0]] + [accs[1 + j] for j in js]
                    res = lax.fori_loop(0, D, dstep, tuple(sel))
                    accs[0] = res[0]
                    for k, j in enumerate(js):
                        accs[1 + j] = res[k + 1]

                for j in range(NEG + 1):
                    plsc.store_scatter(sc_v, [rvec, lanes * 0 + j], accs[j])
                return gcarry

            lax.fori_loop(0, CHUNK // 16, group, 0)
            pltpu.sync_copy(sc_v, out_h.at[pl.ds(base + g * CHUNK, CHUNK)])
            return carry

        lax.fori_loop(0, NCHUNK, chunk, 0)

    return body(cen_idx, ctx_idx, neg_idx, cen_pairs, ctx_pairs)


def _tc_loss(scores):
    def body(s_ref, o_ref):
        x = s_ref[...]
        col = lax.broadcasted_iota(jnp.int32, x.shape, 1)
        valid = col < (NEG + 1)
        xs = jnp.where(valid, x, 0.0)
        sg = jnp.clip(jax.nn.sigmoid(xs), 1e-10, 1.0 - 1e-10)
        contrib = jnp.where(col == 0, -jnp.log(sg), -jnp.log(1.0 - sg))
        contrib = jnp.where(valid, contrib, 0.0)
        o_ref[0, 0] = jnp.sum(contrib) / B

    return pl.pallas_call(
        body,
        out_shape=jax.ShapeDtypeStruct((1, 1), jnp.float32),
        out_specs=pl.BlockSpec(memory_space=pltpu.SMEM),
    )(scores)


@jax.jit
def kernel(center_words, context_words, negative_words, center_emb, context_emb):
    cen_idx = center_words.astype(jnp.int32)
    ctx_idx = context_words.astype(jnp.int32)
    neg_idx = negative_words.astype(jnp.int32).reshape(B * NEG)
    cen_pairs = center_emb.reshape(VOCAB // 2, PAIR)
    ctx_pairs = context_emb.reshape(VOCAB // 2, PAIR)
    scores = _sc_scores(cen_idx, ctx_idx, neg_idx, cen_pairs, ctx_pairs)
    loss = _tc_loss(scores)
    return loss[0, 0]


# speedup vs baseline: 1.0527x; 1.0527x over previous
import functools

import jax
import jax.numpy as jnp
from jax import lax
from jax.experimental import pallas as pl
from jax.experimental.pallas import tpu as pltpu
from jax.experimental.pallas import tpu_sc as plsc

VOCAB = 1000000
D = 64
B = 16384
NEG = 20

NC = 2
NS = 16
NW = NC * NS
BPW = B // NW
CHUNK = 32
NCHUNK = BPW // CHUNK
JH = NEG // 2


def _sc_scores(cen_idx, ctx_idx, neg_idx, cen_emb, ctx_emb):
    mesh = plsc.VectorSubcoreMesh(
        core_axis_name="c", subcore_axis_name="s",
        num_cores=NC, num_subcores=NS,
    )

    @functools.partial(
        pl.kernel,
        out_type=jax.ShapeDtypeStruct((B, 128), jnp.float32),
        mesh=mesh,
        scratch_types=[
            pltpu.VMEM((BPW,), jnp.int32),
            pltpu.VMEM((BPW,), jnp.int32),
            pltpu.VMEM((BPW * NEG,), jnp.int32),
            pltpu.VMEM((CHUNK, D), jnp.float32),
            pltpu.VMEM((CHUNK, D), jnp.float32),
            pltpu.VMEM((CHUNK * NEG, D), jnp.float32),
            pltpu.VMEM((CHUNK, 128), jnp.float32),
            pltpu.SemaphoreType.DMA,
        ],
        compiler_params=pltpu.CompilerParams(
            use_tc_tiling_on_sc=False, needs_layout_passes=False),
    )
    def body(cen_idx_h, ctx_idx_h, neg_idx_h, cen_emb_h, ctx_emb_h, out_h,
             cq_v, xq_v, nq_v, cen_v, ctx_v, neg_v, sc_v, sem):
        wid = lax.axis_index("s") * NC + lax.axis_index("c")
        base = wid * BPW
        pltpu.sync_copy(cen_idx_h.at[pl.ds(base, BPW)], cq_v)
        pltpu.sync_copy(ctx_idx_h.at[pl.ds(base, BPW)], xq_v)
        pltpu.sync_copy(neg_idx_h.at[pl.ds(base * NEG, BPW * NEG)], nq_v)

        lanes = lax.iota(jnp.int32, 16)

        def chunk(g, carry):
            c1 = pltpu.async_copy(
                cen_emb_h.at[cq_v.at[pl.ds(g * CHUNK, CHUNK)]], cen_v, sem)
            c2 = pltpu.async_copy(
                ctx_emb_h.at[xq_v.at[pl.ds(g * CHUNK, CHUNK)]], ctx_v, sem)
            c3 = pltpu.async_copy(
                ctx_emb_h.at[nq_v.at[pl.ds(g * CHUNK * NEG, CHUNK * NEG)]],
                neg_v, sem)
            c1.wait()
            c2.wait()
            c3.wait()

            def group(t, gcarry):
                rvec = t * 16 + lanes
                nrow = rvec * NEG

                accs = [jnp.zeros((16,), jnp.float32) for _ in range(NEG + 1)]
                for half in range(2):
                    js = range(half * JH, (half + 1) * JH)

                    def dstep(d, dc):
                        a = list(dc)
                        dv = lanes * 0 + d
                        c = plsc.load_gather(cen_v, [rvec, dv])
                        if half == 0:
                            x = plsc.load_gather(ctx_v, [rvec, dv])
                            a[0] = a[0] + c * x
                        for k, j in enumerate(js):
                            n = plsc.load_gather(neg_v, [nrow + j, dv])
                            a[k + 1] = a[k + 1] + c * n
                        return tuple(a)

                    sel = [accs[0]] + [accs[1 + j] for j in js]
                    res = lax.fori_loop(0, D, dstep, tuple(sel))
                    accs[0] = res[0]
                    for k, j in enumerate(js):
                        accs[1 + j] = res[k + 1]

                for j in range(NEG + 1):
                    plsc.store_scatter(sc_v, [rvec, lanes * 0 + j], accs[j])
                return gcarry

            lax.fori_loop(0, CHUNK // 16, group, 0)
            pltpu.sync_copy(sc_v, out_h.at[pl.ds(base + g * CHUNK, CHUNK)])
            return carry

        lax.fori_loop(0, NCHUNK, chunk, 0)

    return body(cen_idx, ctx_idx, neg_idx, cen_emb, ctx_emb)


def _tc_loss(scores):
    def body(s_ref, o_ref):
        x = s_ref[...]
        col = lax.broadcasted_iota(jnp.int32, x.shape, 1)
        valid = col < (NEG + 1)
        xs = jnp.where(valid, x, 0.0)
        sg = jnp.clip(jax.nn.sigmoid(xs), 1e-10, 1.0 - 1e-10)
        contrib = jnp.where(col == 0, -jnp.log(sg), -jnp.log(1.0 - sg))
        contrib = jnp.where(valid, contrib, 0.0)
        o_ref[0, 0] = jnp.sum(contrib) / B

    return pl.pallas_call(
        body,
        out_shape=jax.ShapeDtypeStruct((1, 1), jnp.float32),
        out_specs=pl.BlockSpec(memory_space=pltpu.SMEM),
    )(scores)


@jax.jit
def kernel(center_words, context_words, negative_words, center_emb, context_emb):
    cen_idx = center_words.astype(jnp.int32)
    ctx_idx = context_words.astype(jnp.int32)
    neg_idx = negative_words.astype(jnp.int32).reshape(B * NEG)
    scores = _sc_scores(cen_idx, ctx_idx, neg_idx, center_emb, context_emb)
    loss = _tc_loss(scores)
    return loss[0, 0]


# speedup vs baseline: 1.1787x; 1.1197x over previous
import functools

import jax
import jax.numpy as jnp
from jax import lax
from jax.experimental import pallas as pl
from jax.experimental.pallas import tpu as pltpu
from jax.experimental.pallas import tpu_sc as plsc

VOCAB = 1000000
D = 64
B = 16384
NEG = 20

NC = 2
NS = 16
NW = NC * NS
BPW = B // NW
NPW = B * NEG // NW
CN = 640
NCH = NPW // CN


def _sc_gather(cen_idx, ctx_idx, neg_idx, cen_emb, ctx_emb):
    mesh = plsc.VectorSubcoreMesh(
        core_axis_name="c", subcore_axis_name="s",
        num_cores=NC, num_subcores=NS,
    )

    @functools.partial(
        pl.kernel,
        out_type=(
            jax.ShapeDtypeStruct((B, D), jnp.float32),
            jax.ShapeDtypeStruct((B, D), jnp.float32),
            jax.ShapeDtypeStruct((B * NEG, D), jnp.float32),
        ),
        mesh=mesh,
        scratch_types=[
            pltpu.VMEM((BPW,), jnp.int32),
            pltpu.VMEM((BPW,), jnp.int32),
            pltpu.VMEM((NPW,), jnp.int32),
            pltpu.VMEM((BPW, D), jnp.float32),
            pltpu.VMEM((BPW, D), jnp.float32),
            pltpu.VMEM((CN, D), jnp.float32),
            pltpu.SemaphoreType.DMA,
        ],
        compiler_params=pltpu.CompilerParams(
            use_tc_tiling_on_sc=False, needs_layout_passes=False),
    )
    def body(cen_idx_h, ctx_idx_h, neg_idx_h, cen_emb_h, ctx_emb_h,
             cen_out_h, ctx_out_h, neg_out_h,
             cq_v, xq_v, nq_v, cen_v, ctx_v, neg_v, sem):
        wid = lax.axis_index("s") * NC + lax.axis_index("c")
        base = wid * BPW
        nbase = wid * NPW
        pltpu.sync_copy(cen_idx_h.at[pl.ds(base, BPW)], cq_v)
        pltpu.sync_copy(ctx_idx_h.at[pl.ds(base, BPW)], xq_v)
        pltpu.sync_copy(neg_idx_h.at[pl.ds(nbase, NPW)], nq_v)

        c1 = pltpu.async_copy(cen_emb_h.at[cq_v], cen_v, sem)
        c2 = pltpu.async_copy(ctx_emb_h.at[xq_v], ctx_v, sem)
        c1.wait()
        pltpu.sync_copy(cen_v, cen_out_h.at[pl.ds(base, BPW)])
        c2.wait()
        pltpu.sync_copy(ctx_v, ctx_out_h.at[pl.ds(base, BPW)])

        def chunk(g, carry):
            pltpu.async_copy(
                ctx_emb_h.at[nq_v.at[pl.ds(g * CN, CN)]], neg_v, sem).wait()
            pltpu.sync_copy(neg_v, neg_out_h.at[pl.ds(nbase + g * CN, CN)])
            return carry

        lax.fori_loop(0, NCH, chunk, 0)

    return body(cen_idx, ctx_idx, neg_idx, cen_emb, ctx_emb)


BT = 1024
GRID = B // BT


def _tc_loss(cen_g, ctx_g, neg_g):
    def body(cen_ref, ctx_ref, neg_ref, o_ref):
        i = pl.program_id(0)
        cen = cen_ref[...]
        ctx = ctx_ref[...]
        pos = jnp.sum(cen * ctx, axis=1, keepdims=True)
        sgp = jnp.clip(jax.nn.sigmoid(pos), 1e-10, 1.0 - 1e-10)
        total = jnp.sum(-jnp.log(sgp))
        for j in range(NEG):
            nj = neg_ref[j]
            s = jnp.sum(cen * nj, axis=1, keepdims=True)
            sgn = jnp.clip(jax.nn.sigmoid(s), 1e-10, 1.0 - 1e-10)
            total = total + jnp.sum(-jnp.log(1.0 - sgn))

        @pl.when(i == 0)
        def _():
            o_ref[0, 0] = 0.0

        o_ref[0, 0] += total / B

    return pl.pallas_call(
        body,
        grid=(GRID,),
        in_specs=[
            pl.BlockSpec((BT, D), lambda i: (i, 0)),
            pl.BlockSpec((BT, D), lambda i: (i, 0)),
            pl.BlockSpec((NEG, BT, D), lambda i: (0, i, 0)),
        ],
        out_specs=pl.BlockSpec(memory_space=pltpu.SMEM),
        out_shape=jax.ShapeDtypeStruct((1, 1), jnp.float32),
    )(cen_g, ctx_g, neg_g)


@jax.jit
def kernel(center_words, context_words, negative_words, center_emb, context_emb):
    cen_idx = center_words.astype(jnp.int32)
    ctx_idx = context_words.astype(jnp.int32)
    neg_idx = negative_words.astype(jnp.int32).T.reshape(B * NEG)
    cen_g, ctx_g, neg_g = _sc_gather(
        cen_idx, ctx_idx, neg_idx, center_emb, context_emb)
    loss = _tc_loss(cen_g, ctx_g, neg_g.reshape(NEG, B, D))
    return loss[0, 0]


# speedup vs baseline: 1.1836x; 1.0042x over previous
import functools

import jax
import jax.numpy as jnp
from jax import lax
from jax.experimental import pallas as pl
from jax.experimental.pallas import tpu as pltpu
from jax.experimental.pallas import tpu_sc as plsc

VOCAB = 1000000
D = 64
B = 16384
NEG = 20

NC = 2
NS = 16
NW = NC * NS
BPW = B // NW
NPW = B * NEG // NW
CN = 320
NCH = NPW // CN


def _sc_gather(cen_idx, ctx_idx, neg_idx, cen_emb, ctx_emb):
    mesh = plsc.VectorSubcoreMesh(
        core_axis_name="c", subcore_axis_name="s",
        num_cores=NC, num_subcores=NS,
    )

    @functools.partial(
        pl.kernel,
        out_type=(
            jax.ShapeDtypeStruct((B, D), jnp.float32),
            jax.ShapeDtypeStruct((B, D), jnp.float32),
            jax.ShapeDtypeStruct((B * NEG, D), jnp.float32),
        ),
        mesh=mesh,
        scratch_types=[
            pltpu.VMEM((BPW,), jnp.int32),
            pltpu.VMEM((BPW,), jnp.int32),
            pltpu.VMEM((NPW,), jnp.int32),
            pltpu.VMEM((BPW, D), jnp.float32),
            pltpu.VMEM((BPW, D), jnp.float32),
            pltpu.VMEM((CN, D), jnp.float32),
            pltpu.VMEM((CN, D), jnp.float32),
            pltpu.SemaphoreType.DMA,
            pltpu.SemaphoreType.DMA,
        ],
        compiler_params=pltpu.CompilerParams(
            use_tc_tiling_on_sc=False, needs_layout_passes=False),
    )
    def body(cen_idx_h, ctx_idx_h, neg_idx_h, cen_emb_h, ctx_emb_h,
             cen_out_h, ctx_out_h, neg_out_h,
             cq_v, xq_v, nq_v, cen_v, ctx_v, neg_v0, neg_v1, sem, osem):
        wid = lax.axis_index("s") * NC + lax.axis_index("c")
        base = wid * BPW
        nbase = wid * NPW
        pltpu.sync_copy(cen_idx_h.at[pl.ds(base, BPW)], cq_v)
        pltpu.sync_copy(ctx_idx_h.at[pl.ds(base, BPW)], xq_v)
        pltpu.sync_copy(neg_idx_h.at[pl.ds(nbase, NPW)], nq_v)

        c1 = pltpu.async_copy(cen_emb_h.at[cq_v], cen_v, sem)
        c2 = pltpu.async_copy(ctx_emb_h.at[xq_v], ctx_v, sem)
        c1.wait()
        pltpu.sync_copy(cen_v, cen_out_h.at[pl.ds(base, BPW)])
        c2.wait()
        pltpu.sync_copy(ctx_v, ctx_out_h.at[pl.ds(base, BPW)])

        def pair(t, carry):
            a = 2 * t
            b = 2 * t + 1
            ga = pltpu.async_copy(
                ctx_emb_h.at[nq_v.at[pl.ds(a * CN, CN)]], neg_v0, sem)
            gb = pltpu.async_copy(
                ctx_emb_h.at[nq_v.at[pl.ds(b * CN, CN)]], neg_v1, sem)
            ga.wait()
            oa = pltpu.async_copy(
                neg_v0, neg_out_h.at[pl.ds(nbase + a * CN, CN)], osem)
            gb.wait()
            ob = pltpu.async_copy(
                neg_v1, neg_out_h.at[pl.ds(nbase + b * CN, CN)], osem)
            oa.wait()
            ob.wait()
            return carry

        lax.fori_loop(0, NCH // 2, pair, 0)

    return body(cen_idx, ctx_idx, neg_idx, cen_emb, ctx_emb)


BT = 1024
GRID = B // BT


def _tc_loss(cen_g, ctx_g, neg_g):
    def body(cen_ref, ctx_ref, neg_ref, o_ref):
        i = pl.program_id(0)
        cen = cen_ref[...]
        ctx = ctx_ref[...]
        pos = jnp.sum(cen * ctx, axis=1, keepdims=True)
        sgp = jnp.clip(jax.nn.sigmoid(pos), 1e-10, 1.0 - 1e-10)
        total = jnp.sum(-jnp.log(sgp))
        for j in range(NEG):
            nj = neg_ref[j]
            s = jnp.sum(cen * nj, axis=1, keepdims=True)
            sgn = jnp.clip(jax.nn.sigmoid(s), 1e-10, 1.0 - 1e-10)
            total = total + jnp.sum(-jnp.log(1.0 - sgn))

        @pl.when(i == 0)
        def _():
            o_ref[0, 0] = 0.0

        o_ref[0, 0] += total / B

    return pl.pallas_call(
        body,
        grid=(GRID,),
        in_specs=[
            pl.BlockSpec((BT, D), lambda i: (i, 0)),
            pl.BlockSpec((BT, D), lambda i: (i, 0)),
            pl.BlockSpec((NEG, BT, D), lambda i: (0, i, 0)),
        ],
        out_specs=pl.BlockSpec(memory_space=pltpu.SMEM),
        out_shape=jax.ShapeDtypeStruct((1, 1), jnp.float32),
    )(cen_g, ctx_g, neg_g)


@jax.jit
def kernel(center_words, context_words, negative_words, center_emb, context_emb):
    cen_idx = center_words.astype(jnp.int32)
    ctx_idx = context_words.astype(jnp.int32)
    neg_idx = negative_words.astype(jnp.int32).T.reshape(B * NEG)
    cen_g, ctx_g, neg_g = _sc_gather(
        cen_idx, ctx_idx, neg_idx, center_emb, context_emb)
    loss = _tc_loss(cen_g, ctx_g, neg_g.reshape(NEG, B, D))
    return loss[0, 0]


# speedup vs baseline: 1.1862x; 1.0022x over previous
import functools

import jax
import jax.numpy as jnp
from jax import lax
from jax.experimental import pallas as pl
from jax.experimental.pallas import tpu as pltpu
from jax.experimental.pallas import tpu_sc as plsc

VOCAB = 1000000
D = 64
B = 16384
NEG = 20

NC = 2
NS = 16
NW = NC * NS
CN = 320
BT = 1024


@functools.lru_cache(maxsize=None)
def _sc_gather_fn(n):
    bpw = n // NW
    npw = n * NEG // NW
    nch = npw // CN
    mesh = plsc.VectorSubcoreMesh(
        core_axis_name="c", subcore_axis_name="s",
        num_cores=NC, num_subcores=NS,
    )

    @functools.partial(
        pl.kernel,
        out_type=(
            jax.ShapeDtypeStruct((n, D), jnp.float32),
            jax.ShapeDtypeStruct((n, D), jnp.float32),
            jax.ShapeDtypeStruct((n * NEG, D), jnp.float32),
        ),
        mesh=mesh,
        scratch_types=[
            pltpu.VMEM((bpw,), jnp.int32),
            pltpu.VMEM((bpw,), jnp.int32),
            pltpu.VMEM((npw,), jnp.int32),
            pltpu.VMEM((bpw, D), jnp.float32),
            pltpu.VMEM((bpw, D), jnp.float32),
            pltpu.VMEM((CN, D), jnp.float32),
            pltpu.VMEM((CN, D), jnp.float32),
            pltpu.SemaphoreType.DMA,
            pltpu.SemaphoreType.DMA,
        ],
        compiler_params=pltpu.CompilerParams(
            use_tc_tiling_on_sc=False, needs_layout_passes=False),
    )
    def body(cen_idx_h, ctx_idx_h, neg_idx_h, cen_emb_h, ctx_emb_h,
             cen_out_h, ctx_out_h, neg_out_h,
             cq_v, xq_v, nq_v, cen_v, ctx_v, neg_v0, neg_v1, sem, osem):
        wid = lax.axis_index("s") * NC + lax.axis_index("c")
        base = wid * bpw
        nbase = wid * npw
        pltpu.sync_copy(cen_idx_h.at[pl.ds(base, bpw)], cq_v)
        pltpu.sync_copy(ctx_idx_h.at[pl.ds(base, bpw)], xq_v)
        pltpu.sync_copy(neg_idx_h.at[pl.ds(nbase, npw)], nq_v)

        c1 = pltpu.async_copy(cen_emb_h.at[cq_v], cen_v, sem)
        c2 = pltpu.async_copy(ctx_emb_h.at[xq_v], ctx_v, sem)
        c1.wait()
        pltpu.sync_copy(cen_v, cen_out_h.at[pl.ds(base, bpw)])
        c2.wait()
        pltpu.sync_copy(ctx_v, ctx_out_h.at[pl.ds(base, bpw)])

        def pair(t, carry):
            a = 2 * t
            b = 2 * t + 1
            ga = pltpu.async_copy(
                ctx_emb_h.at[nq_v.at[pl.ds(a * CN, CN)]], neg_v0, sem)
            gb = pltpu.async_copy(
                ctx_emb_h.at[nq_v.at[pl.ds(b * CN, CN)]], neg_v1, sem)
            ga.wait()
            oa = pltpu.async_copy(
                neg_v0, neg_out_h.at[pl.ds(nbase + a * CN, CN)], osem)
            gb.wait()
            ob = pltpu.async_copy(
                neg_v1, neg_out_h.at[pl.ds(nbase + b * CN, CN)], osem)
            oa.wait()
            ob.wait()
            return carry

        lax.fori_loop(0, nch // 2, pair, 0)

    return body


def _tc_loss(cen_g, ctx_g, neg_g):
    n = cen_g.shape[0]

    def body(cen_ref, ctx_ref, neg_ref, o_ref):
        i = pl.program_id(0)
        cen = cen_ref[...]
        ctx = ctx_ref[...]
        pos = jnp.sum(cen * ctx, axis=1, keepdims=True)
        sgp = jnp.clip(jax.nn.sigmoid(pos), 1e-10, 1.0 - 1e-10)
        total = jnp.sum(-jnp.log(sgp))
        for j in range(NEG):
            nj = neg_ref[j]
            s = jnp.sum(cen * nj, axis=1, keepdims=True)
            sgn = jnp.clip(jax.nn.sigmoid(s), 1e-10, 1.0 - 1e-10)
            total = total + jnp.sum(-jnp.log(1.0 - sgn))

        @pl.when(i == 0)
        def _():
            o_ref[0, 0] = 0.0

        o_ref[0, 0] += total / B

    return pl.pallas_call(
        body,
        grid=(n // BT,),
        in_specs=[
            pl.BlockSpec((BT, D), lambda i: (i, 0)),
            pl.BlockSpec((BT, D), lambda i: (i, 0)),
            pl.BlockSpec((NEG, BT, D), lambda i: (0, i, 0)),
        ],
        out_specs=pl.BlockSpec(memory_space=pltpu.SMEM),
        out_shape=jax.ShapeDtypeStruct((1, 1), jnp.float32),
    )(cen_g, ctx_g, neg_g)


def _half(cen_idx, ctx_idx, neg_jm, cen_emb, ctx_emb):
    n = cen_idx.shape[0]
    cen_g, ctx_g, neg_g = _sc_gather_fn(n)(
        cen_idx, ctx_idx, neg_jm.reshape(n * NEG), cen_emb, ctx_emb)
    return _tc_loss(cen_g, ctx_g, neg_g.reshape(NEG, n, D))


@jax.jit
def kernel(center_words, context_words, negative_words, center_emb, context_emb):
    cen_idx = center_words.astype(jnp.int32)
    ctx_idx = context_words.astype(jnp.int32)
    neg_jm = negative_words.astype(jnp.int32).T
    h = B // 2
    l1 = _half(cen_idx[:h], ctx_idx[:h], neg_jm[:, :h], center_emb, context_emb)
    l2 = _half(cen_idx[h:], ctx_idx[h:], neg_jm[:, h:], center_emb, context_emb)
    return (l1 + l2)[0, 0]
